# baseline (speedup 1.0000x reference)
import jax
import jax.numpy as jnp
from jax import lax
from jax.experimental import pallas as pl
from jax.experimental.pallas import tpu as pltpu
from jax.experimental.pallas import tpu_sc as plsc

DIM = 64
B0, B1 = 16384, 26
B_TOTAL = B0 * B1
NUM_WORKERS = 32
CHUNK = 128
N_JOBS = B_TOTAL // CHUNK
JOBS_PER_W = N_JOBS // NUM_WORKERS
PER_W = JOBS_PER_W * CHUNK
N_PAIRS = JOBS_PER_W // 2
BH = B0 // CHUNK


def _gather_body(table_hbm, idx_hbm, out_hbm, idx_v, g0, g1, tr, sem0, sem1):
    wid = lax.axis_index("s") * 2 + lax.axis_index("c")
    jbase = wid * JOBS_PER_W
    ibase = pl.multiple_of(jbase * CHUNK, PER_W)
    pltpu.sync_copy(idx_hbm.at[pl.ds(ibase, PER_W)], idx_v)

    iota = lax.iota(jnp.int32, 16)
    bvecs = [iota + (16 * k) for k in range(8)]

    def start_gather(jj, g, sem):
        off = pl.multiple_of(jj * CHUNK, CHUNK)
        pltpu.async_copy(table_hbm.at[idx_v.at[pl.ds(off, CHUNK)]], g, sem)

    def wait_gather(g, sem):
        pltpu.make_async_copy(table_hbm.at[pl.ds(0, CHUNK)], g, sem).wait()

    def transpose_write(jj, g):
        j = jbase + jj
        t = j // BH
        bh = j % BH
        for d in range(DIM):
            dvec = jnp.broadcast_to(jnp.int32(d), (16,))
            for k in range(8):
                v = plsc.load_gather(g, [bvecs[k], dvec])
                tr[d // 8, d % 8, pl.ds(16 * k, 16)] = v
        pltpu.sync_copy(tr, out_hbm.at[t, :, bh])

    start_gather(0, g0, sem0)

    def pair(p, carry):
        j0 = p * 2
        start_gather(j0 + 1, g1, sem1)
        wait_gather(g0, sem0)
        transpose_write(j0, g0)

        @pl.when(p < N_PAIRS - 1)
        def _():
            start_gather(j0 + 2, g0, sem0)

        wait_gather(g1, sem1)
        transpose_write(j0 + 1, g1)
        return carry

    lax.fori_loop(0, N_PAIRS, pair, 0)


def kernel(X, embeddings):
    idx = X.T.reshape(-1)
    mesh = plsc.VectorSubcoreMesh(core_axis_name="c", subcore_axis_name="s")
    out5 = pl.kernel(
        _gather_body,
        out_type=jax.ShapeDtypeStruct((B1, DIM // 8, BH, 8, CHUNK), jnp.float32),
        mesh=mesh,
        scratch_types=[
            pltpu.VMEM((PER_W,), jnp.int32),
            pltpu.VMEM((CHUNK, DIM), jnp.float32),
            pltpu.VMEM((CHUNK, DIM), jnp.float32),
            pltpu.VMEM((DIM // 8, 8, CHUNK), jnp.float32),
            pltpu.SemaphoreType.DMA,
            pltpu.SemaphoreType.DMA,
        ],
        compiler_params=pltpu.CompilerParams(
            use_tc_tiling_on_sc=False, needs_layout_passes=False
        ),
    )(embeddings, idx)
    return out5.transpose(2, 4, 0, 1, 3).reshape(B0, B1, DIM)


# speedup vs baseline: 1.4338x; 1.4338x over previous
import functools

import jax
import jax.numpy as jnp
from jax import lax
from jax.experimental import pallas as pl
from jax.experimental.pallas import tpu as pltpu
from jax.experimental.pallas import tpu_sc as plsc

DIM = 64
B0, B1 = 16384, 26
B_TOTAL = B0 * B1
NUM_WORKERS = 32
PER_W = B_TOTAL // NUM_WORKERS
CHUNK = 512
N_CHUNKS = PER_W // CHUNK
N_PAIRS = N_CHUNKS // 2


def _gather_body(table_hbm, idx_hbm, out_hbm, idx_v, rows0, rows1, sem0, sem1):
    wid = lax.axis_index("s") * 2 + lax.axis_index("c")
    base = pl.multiple_of(wid * PER_W, PER_W)
    pltpu.sync_copy(idx_hbm.at[pl.ds(base, PER_W)], idx_v)

    def start_gather(i, rows, sem):
        off = pl.multiple_of(i * CHUNK, CHUNK)
        pltpu.async_copy(table_hbm.at[idx_v.at[pl.ds(off, CHUNK)]], rows, sem)

    def wait_gather(rows, sem):
        pltpu.make_async_copy(table_hbm.at[pl.ds(0, CHUNK)], rows, sem).wait()

    def write(i, rows):
        off = pl.multiple_of(i * CHUNK, CHUNK)
        pltpu.sync_copy(rows, out_hbm.at[pl.ds(base + off, CHUNK)])

    start_gather(0, rows0, sem0)

    def pair_body(p, carry):
        i0 = p * 2
        start_gather(i0 + 1, rows1, sem1)
        wait_gather(rows0, sem0)
        write(i0, rows0)

        @pl.when(p < N_PAIRS - 1)
        def _():
            start_gather(i0 + 2, rows0, sem0)

        wait_gather(rows1, sem1)
        write(i0 + 1, rows1)
        return carry

    lax.fori_loop(0, N_PAIRS, pair_body, 0)


def kernel(X, embeddings):
    idx = X.reshape(-1)
    mesh = plsc.VectorSubcoreMesh(core_axis_name="c", subcore_axis_name="s")
    out = pl.kernel(
        _gather_body,
        out_type=jax.ShapeDtypeStruct((B_TOTAL, DIM), jnp.float32),
        mesh=mesh,
        scratch_types=[
            pltpu.VMEM((PER_W,), jnp.int32),
            pltpu.VMEM((CHUNK, DIM), jnp.float32),
            pltpu.VMEM((CHUNK, DIM), jnp.float32),
            pltpu.SemaphoreType.DMA,
            pltpu.SemaphoreType.DMA,
        ],
        compiler_params=pltpu.CompilerParams(use_tc_tiling_on_sc=False),
    )(embeddings, idx)
    return out.reshape(B0, B1, DIM)
